# baseline (speedup 1.0000x reference)
import jax
import jax.numpy as jnp
from jax import lax
from jax.experimental import pallas as pl
from jax.experimental.pallas import tpu as pltpu
from jax.experimental.pallas import tpu_sc as plsc

NC = 2
NS = 16
NW = NC * NS
L = 16

HER_PROPORTION = 0.8
THRESHOLD = 0.05
TH_SQ = THRESHOLD * THRESHOLD


def _her_body(ach_hbm, des_hbm, rew_hbm, buf_hbm, noise_hbm, idx_hbm,
              goal_out, rew_out,
              idx_v, fut_v, ach_v, des_v, noise_v, rew_v, rewo_v, scr_v,
              gsem):
    B, T, D = ach_hbm.shape
    bpw = B // NW
    ngrp = bpw // L

    wid = lax.axis_index("s") * NC + lax.axis_index("c")
    base = wid * bpw

    pltpu.sync_copy(idx_hbm.at[pl.ds(base, bpw)], idx_v)
    gather = pltpu.async_copy(buf_hbm.at[idx_v], fut_v, gsem)
    pltpu.sync_copy(ach_hbm.at[pl.ds(base, bpw)], ach_v)
    pltpu.sync_copy(des_hbm.at[pl.ds(base, bpw)], des_v)
    pltpu.sync_copy(noise_hbm.at[pl.ds(base, bpw)], noise_v)
    pltpu.sync_copy(rew_hbm.at[pl.ds(base, bpw)], rew_v)
    gather.wait()

    lane = lax.iota(jnp.int32, L)
    lane32 = lane * (2 * L)
    l2 = lane // 2
    lpar = lane % 2

    def group(g, carry):
        r0 = g * L
        nz = noise_v[pl.ds(r0, L)]
        for k in range(L):
            r = r0 + k
            cond = nz[k] < HER_PROPORTION
            for t in range(T):
                acc = jnp.zeros((L,), jnp.float32)
                for j in range(D // L):
                    a = ach_v[r, t, pl.ds(j * L, L)]
                    d = des_v[r, t, pl.ds(j * L, L)]
                    f = fut_v[r, pl.ds(j * L, L)]
                    gsel = jnp.where(cond, f, d)
                    des_v[r, t, pl.ds(j * L, L)] = gsel
                    diff = a - gsel
                    acc = acc + diff * diff
                plsc.store_scatter(scr_v, [lane32 + (T * k + t)], acc)
        for h in range(T):
            rows = r0 + h * (L // 2) + l2
            tot = scr_v[pl.ds(h * L, L)]
            for i in range(1, L):
                tot = tot + scr_v[pl.ds(i * (2 * L) + h * L, L)]
            nr = -(tot >= TH_SQ).astype(jnp.float32)
            cdup = plsc.load_gather(noise_v, [rows]) < HER_PROPORTION
            rw = plsc.load_gather(rew_v, [rows, lpar])
            plsc.store_scatter(rewo_v, [rows, lpar],
                               jnp.where(cdup, nr, rw))
        return carry

    lax.fori_loop(0, ngrp, group, 0)

    pltpu.sync_copy(des_v, goal_out.at[pl.ds(base, bpw)])
    pltpu.sync_copy(rewo_v, rew_out.at[pl.ds(base, bpw)])


def kernel(achieved_goal, desired_goal, reward, buffer_ag, her_noise, future_idx):
    B, T, D = achieved_goal.shape
    bpw = B // NW
    idx32 = future_idx.astype(jnp.int32)

    mesh = plsc.VectorSubcoreMesh(core_axis_name="c", subcore_axis_name="s",
                                  num_cores=NC, num_subcores=NS)
    run = pl.kernel(
        _her_body,
        out_type=(
            jax.ShapeDtypeStruct((B, T, D), jnp.float32),
            jax.ShapeDtypeStruct((B, T), jnp.float32),
        ),
        mesh=mesh,
        compiler_params=pltpu.CompilerParams(needs_layout_passes=False,
                                             use_tc_tiling_on_sc=False),
        scratch_types=[
            pltpu.VMEM((bpw,), jnp.int32),
            pltpu.VMEM((bpw, D), jnp.float32),
            pltpu.VMEM((bpw, T, D), jnp.float32),
            pltpu.VMEM((bpw, T, D), jnp.float32),
            pltpu.VMEM((bpw,), jnp.float32),
            pltpu.VMEM((bpw, T), jnp.float32),
            pltpu.VMEM((bpw, T), jnp.float32),
            pltpu.VMEM((2 * L * L,), jnp.float32),
            pltpu.SemaphoreType.DMA,
        ],
    )
    goal, rew = run(achieved_goal, desired_goal, reward, buffer_ag,
                    her_noise, idx32)
    return goal, rew


# speedup vs baseline: 1.0194x; 1.0194x over previous
import jax
import jax.numpy as jnp
from jax import lax
from jax.experimental import pallas as pl
from jax.experimental.pallas import tpu as pltpu
from jax.experimental.pallas import tpu_sc as plsc

NC = 2
NS = 16
NW = NC * NS
L = 16
BW = 128

HER_PROPORTION = 0.8
THRESHOLD = 0.05
TH_SQ = THRESHOLD * THRESHOLD


def _her_body(ach_hbm, des_hbm, rew_hbm, buf_hbm, noise_hbm, idx_hbm,
              goal_out, rew_out,
              idx_v, fut_v, ach_v, des_v, noise_v, rew_v, rewo_v, gsem):
    T = ach_hbm.shape[0]
    D = ach_hbm.shape[1] * ach_hbm.shape[3]
    NCH = BW // L

    wid = lax.axis_index("s") * NC + lax.axis_index("c")
    base = wid * BW

    pltpu.sync_copy(idx_hbm.at[pl.ds(base, BW)], idx_v)
    gather = pltpu.async_copy(buf_hbm.at[idx_v], fut_v, gsem)
    for t in range(T):
        for r in range(D // 8):
            pltpu.sync_copy(ach_hbm.at[t, r, wid], ach_v.at[t, pl.ds(r * 8, 8)])
            pltpu.sync_copy(des_hbm.at[t, r, wid], des_v.at[t, pl.ds(r * 8, 8)])
    pltpu.sync_copy(noise_hbm.at[pl.ds(base, BW)], noise_v)
    pltpu.sync_copy(rew_hbm.at[wid], rew_v)
    gather.wait()

    lane = lax.iota(jnp.int32, L)
    conds = [noise_v[pl.ds(i * L, L)] < HER_PROPORTION for i in range(NCH)]
    rows = [lane + i * L for i in range(NCH)]

    def dstep(d, accs):
        dvec = jnp.broadcast_to(d, (L,))
        out = []
        for i in range(NCH):
            futcol = plsc.load_gather(fut_v, [rows[i], dvec])
            for t in range(T):
                a = ach_v[t, d, pl.ds(i * L, L)]
                de = des_v[t, d, pl.ds(i * L, L)]
                g = jnp.where(conds[i], futcol, de)
                des_v[t, d, pl.ds(i * L, L)] = g
                diff = a - g
                out.append(accs[t * NCH + i] + diff * diff)
        return [out[i * T + t] for t in range(T) for i in range(NCH)]

    accs = lax.fori_loop(0, D, dstep,
                         [jnp.zeros((L,), jnp.float32)] * (T * NCH))

    for t in range(T):
        for i in range(NCH):
            nr = -(accs[t * NCH + i] >= TH_SQ).astype(jnp.float32)
            rw = rew_v[t, pl.ds(i * L, L)]
            rewo_v[t, pl.ds(i * L, L)] = jnp.where(conds[i], nr, rw)

    for t in range(T):
        for r in range(D // 8):
            pltpu.sync_copy(des_v.at[t, pl.ds(r * 8, 8)], goal_out.at[t, r, wid])
    pltpu.sync_copy(rewo_v, rew_out.at[wid])


def kernel(achieved_goal, desired_goal, reward, buffer_ag, her_noise, future_idx):
    B, T, D = achieved_goal.shape
    idx32 = future_idx.astype(jnp.int32)

    def to5(x):
        return (x.transpose(1, 2, 0)
                 .reshape(T, D // 8, 8, B // 128, 128)
                 .transpose(0, 1, 3, 2, 4))

    ach5 = to5(achieved_goal)
    des5 = to5(desired_goal)
    rew3 = reward.reshape(B // 128, 128, T).transpose(0, 2, 1)

    mesh = plsc.VectorSubcoreMesh(core_axis_name="c", subcore_axis_name="s",
                                  num_cores=NC, num_subcores=NS)
    run = pl.kernel(
        _her_body,
        out_type=(
            jax.ShapeDtypeStruct((T, D // 8, B // 128, 8, 128), jnp.float32),
            jax.ShapeDtypeStruct((B // 128, T, 128), jnp.float32),
        ),
        mesh=mesh,
        compiler_params=pltpu.CompilerParams(needs_layout_passes=False,
                                             use_tc_tiling_on_sc=False),
        scratch_types=[
            pltpu.VMEM((BW,), jnp.int32),
            pltpu.VMEM((BW, D), jnp.float32),
            pltpu.VMEM((T, D, 128), jnp.float32),
            pltpu.VMEM((T, D, 128), jnp.float32),
            pltpu.VMEM((BW,), jnp.float32),
            pltpu.VMEM((T, 128), jnp.float32),
            pltpu.VMEM((T, 128), jnp.float32),
            pltpu.SemaphoreType.DMA,
        ],
    )
    goal5, rew3o = run(ach5, des5, rew3, buffer_ag, her_noise, idx32)

    goal = (goal5.transpose(0, 1, 3, 2, 4)
                 .reshape(T, D, B)
                 .transpose(2, 0, 1))
    rew = rew3o.transpose(0, 2, 1).reshape(B, T)
    return goal, rew


# speedup vs baseline: 1.2027x; 1.1797x over previous
import jax
import jax.numpy as jnp
from jax import lax
from jax.experimental import pallas as pl
from jax.experimental.pallas import tpu as pltpu
from jax.experimental.pallas import tpu_sc as plsc

NC = 2
NS = 16
NW = NC * NS
L = 16
BW = 128

HER_PROPORTION = 0.8
THRESHOLD = 0.05
TH_SQ = THRESHOLD * THRESHOLD


def _her_body(ach_hbm, des_hbm, rew_hbm, buf_hbm, noise_hbm, idx_hbm,
              goal_out, rew_out,
              idx_v, idx2_v, fut_v, ach_v, des_v, noise_v, rew_v, rewo_v,
              gsem, dsem, osem):
    T = ach_hbm.shape[0]
    D = ach_hbm.shape[1] * ach_hbm.shape[3]
    NCH = BW // L

    wid = lax.axis_index("s") * NC + lax.axis_index("c")
    base = wid * BW

    pltpu.sync_copy(idx_hbm.at[pl.ds(base, BW)], idx_v)

    lane = lax.iota(jnp.int32, L)
    for i in range(NCH):
        iv = idx_v[pl.ds(i * L, L)]
        plsc.store_scatter(idx2_v, [lane + i * L],
                           lax.shift_right_logical(iv, 1))
    gather = pltpu.async_copy(buf_hbm.at[idx2_v], fut_v, gsem)

    dense = []
    for t in range(T):
        for r in range(D // 8):
            dense.append(pltpu.async_copy(
                ach_hbm.at[t, r, wid], ach_v.at[t, pl.ds(r * 8, 8)], dsem))
            dense.append(pltpu.async_copy(
                des_hbm.at[t, r, wid], des_v.at[t, pl.ds(r * 8, 8)], dsem))
    dense.append(pltpu.async_copy(noise_hbm.at[pl.ds(base, BW)], noise_v, dsem))
    dense.append(pltpu.async_copy(rew_hbm.at[pl.ds(wid * T, T)], rew_v, dsem))
    for c in dense:
        c.wait()
    gather.wait()

    for i in range(NCH):
        cond = noise_v[pl.ds(i * L, L)] < HER_PROPORTION
        rows = lane + i * L
        par = (idx_v[pl.ds(i * L, L)] & 1) * D
        accs = [jnp.zeros((L,), jnp.float32) for _ in range(T)]

        def dstep(d, accs, cond=cond, rows=rows, par=par, i=i):
            fut = plsc.load_gather(fut_v, [rows, par + d])
            out = []
            for t in range(T):
                a = ach_v[t, d, pl.ds(i * L, L)]
                de = des_v[t, d, pl.ds(i * L, L)]
                g = jnp.where(cond, fut, de)
                des_v[t, d, pl.ds(i * L, L)] = g
                diff = a - g
                out.append(accs[t] + diff * diff)
            return out

        accs = lax.fori_loop(0, D, dstep, accs, unroll=4)
        for t in range(T):
            nr = -(accs[t] >= TH_SQ).astype(jnp.float32)
            rw = rew_v[t, pl.ds(i * L, L)]
            rewo_v[t, pl.ds(i * L, L)] = jnp.where(cond, nr, rw)

    outs = []
    for t in range(T):
        for r in range(D // 8):
            outs.append(pltpu.async_copy(
                des_v.at[t, pl.ds(r * 8, 8)], goal_out.at[t, r, wid], osem))
    outs.append(pltpu.async_copy(rewo_v, rew_out.at[pl.ds(wid * T, T)], osem))
    for c in outs:
        c.wait()


def kernel(achieved_goal, desired_goal, reward, buffer_ag, her_noise, future_idx):
    B, T, D = achieved_goal.shape
    BUF = buffer_ag.shape[0]
    idx32 = future_idx.astype(jnp.int32)

    def to5(x):
        return (x.transpose(1, 2, 0)
                 .reshape(T, D // 8, 8, B // 128, 128)
                 .transpose(0, 1, 3, 2, 4))

    ach5 = to5(achieved_goal)
    des5 = to5(desired_goal)
    rew2 = (reward.reshape(B // 128, 128, T)
                  .transpose(0, 2, 1)
                  .reshape(B // 128 * T, 128))
    buf2 = buffer_ag.reshape(BUF // 2, 2 * D)

    mesh = plsc.VectorSubcoreMesh(core_axis_name="c", subcore_axis_name="s",
                                  num_cores=NC, num_subcores=NS)
    run = pl.kernel(
        _her_body,
        out_type=(
            jax.ShapeDtypeStruct((T, D // 8, B // 128, 8, 128), jnp.float32),
            jax.ShapeDtypeStruct((B // 128 * T, 128), jnp.float32),
        ),
        mesh=mesh,
        compiler_params=pltpu.CompilerParams(needs_layout_passes=False,
                                             use_tc_tiling_on_sc=True),
        scratch_types=[
            pltpu.VMEM((BW,), jnp.int32),
            pltpu.VMEM((BW,), jnp.int32),
            pltpu.VMEM((BW, 2 * D), jnp.float32),
            pltpu.VMEM((T, D, 128), jnp.float32),
            pltpu.VMEM((T, D, 128), jnp.float32),
            pltpu.VMEM((BW,), jnp.float32),
            pltpu.VMEM((T, 128), jnp.float32),
            pltpu.VMEM((T, 128), jnp.float32),
            pltpu.SemaphoreType.DMA,
            pltpu.SemaphoreType.DMA,
            pltpu.SemaphoreType.DMA,
        ],
    )
    goal5, rew2o = run(ach5, des5, rew2, buf2, her_noise, idx32)

    goal = (goal5.transpose(0, 1, 3, 2, 4)
                 .reshape(T, D, B)
                 .transpose(2, 0, 1))
    rew = (rew2o.reshape(B // 128, T, 128)
                .transpose(0, 2, 1)
                .reshape(B, T))
    return goal, rew
